# baseline (speedup 1.0000x reference)
import functools

import jax
import jax.numpy as jnp
from jax import lax
from jax.experimental import pallas as pl
from jax.experimental.pallas import tpu as pltpu
from jax.experimental.pallas import tpu_sc as plsc

B = 16
N = 10000
E = 320000
NN = B * N
BE = B * E
NUM_PASSES = 4

K = 6400
CPW = K // 128
NWIN = E // K
NV = N // 16
ES = E // 16


def _fused_body(x_hbm, pk_hbm, ew_hbm, ewm_hbm, thr_hbm, fc_hbm, out_hbm,
                wc_sp, mm_sp,
                x_t, aggr_t, thr_t, stage, stage2,
                pk0, pk1, w0, w1,
                mmstage, mmall, fcbuf, headbuf,
                semS0, semS1, semW0, semW1, sem_x):
    t = lax.axis_index("s")
    tN = t * N

    pltpu.async_copy(x_hbm.at[pl.ds(tN, N)], x_t, sem_x).wait()
    pltpu.async_copy(thr_hbm.at[pl.ds(0, N)], thr_t, sem_x).wait()
    pltpu.async_copy(ew_hbm.at[pl.ds(t * ES, ES)], stage, sem_x).wait()
    pltpu.async_copy(ewm_hbm.at[pl.ds(t * ES, ES)], stage2, sem_x).wait()

    @plsc.parallel_loop(0, ES // 16, step=1, unroll=5)
    def wc_body(i):
        stage[pl.ds(i * 16, 16)] = (stage[pl.ds(i * 16, 16)]
                                    * stage2[pl.ds(i * 16, 16)])

    pltpu.async_copy(stage.at[pl.ds(0, ES)],
                     wc_sp.at[pl.ds(t * ES, ES)], sem_x).wait()
    pltpu.async_copy(fc_hbm.at[pl.ds(0, 16)], fcbuf, sem_x).wait()
    plsc.subcore_barrier()

    zeros16 = jnp.zeros((16,), jnp.float32)

    def edge_window(k, pkb, wb, semS, semW, prefetch):
        pltpu.make_async_copy(pk_hbm.at[pl.ds(0, K)], pkb, semS).wait()
        pltpu.make_async_copy(ew_hbm.at[pl.ds(0, K)], wb, semW).wait()

        @plsc.parallel_loop(0, K // 16, step=1, unroll=40)
        def vec_body(i):
            pk = pkb[pl.ds(i * 16, 16)]
            sl = lax.shift_right_logical(pk, 14)
            dl = lax.bitwise_and(pk, 0x3FFF)
            xs = plsc.load_gather(x_t, [sl])
            plsc.addupdate_scatter(aggr_t, [dl],
                                   xs * wb[pl.ds(i * 16, 16)])

        def fire():
            pltpu.async_copy(pk_hbm.at[pl.ds(t * E + (k + 2) * K, K)],
                             pkb, semS)
            pltpu.async_copy(wc_sp.at[pl.ds((k + 2) * K, K)], wb, semW)
        if prefetch is True:
            fire()
        elif prefetch is not False:
            pl.when(prefetch)(fire)

    def one_pass():
        def zero_body(i, _):
            for u in range(5):
                aggr_t[pl.ds((i * 5 + u) * 16, 16)] = zeros16
            return 0
        lax.fori_loop(0, NV // 5, zero_body, 0)

        pltpu.async_copy(pk_hbm.at[pl.ds(t * E, K)], pk0, semS0)
        pltpu.async_copy(wc_sp.at[pl.ds(0, K)], w0, semW0)
        pltpu.async_copy(pk_hbm.at[pl.ds(t * E + K, K)], pk1, semS1)
        pltpu.async_copy(wc_sp.at[pl.ds(K, K)], w1, semW1)

        edge_window(0, pk0, w0, semS0, semW0, True)
        edge_window(1, pk1, w1, semS1, semW1, True)

        def pair_body(m, _):
            pf = m < (NWIN // 2 - 1)
            edge_window(2 * m, pk0, w0, semS0, semW0, pf)
            edge_window(2 * m + 1, pk1, w1, semS1, semW1, pf)
            return 0
        lax.fori_loop(1, NWIN // 2, pair_body, 0)

        def mm_body(i, carry):
            mn, mx = carry
            for u in range(5):
                a = aggr_t[pl.ds((i * 5 + u) * 16, 16)]
                mn = jnp.minimum(mn, a)
                mx = jnp.maximum(mx, a)
            return mn, mx
        mnv, mxv = lax.fori_loop(
            0, NV // 5, mm_body,
            (jnp.full((16,), jnp.inf, jnp.float32),
             jnp.full((16,), -jnp.inf, jnp.float32)))
        mmstage[0, pl.ds(0, 16)] = mnv
        mmstage[1, pl.ds(0, 16)] = mxv
        pltpu.async_copy(mmstage.at[pl.ds(0, 2)], mm_sp.at[t], sem_x).wait()
        plsc.subcore_barrier()
        pltpu.async_copy(mm_sp.at[pl.ds(0, 16)], mmall, sem_x).wait()
        gmn = jnp.full((16,), jnp.inf, jnp.float32)
        gmx = jnp.full((16,), -jnp.inf, jnp.float32)
        for r in range(16):
            gmn = jnp.minimum(gmn, mmall[r, 0, pl.ds(0, 16)])
            gmx = jnp.maximum(gmx, mmall[r, 1, pl.ds(0, 16)])
        mnb = jnp.broadcast_to(jnp.min(gmn), (16,))
        mxb = jnp.broadcast_to(jnp.max(gmx), (16,))
        invb = 1.0 / (mxb - mnb)

        def upd_body(i, _):
            for u in range(5):
                j = (i * 5 + u) * 16
                a = aggr_t[pl.ds(j, 16)]
                z = (a - mnb) * invb - jnp.abs(thr_t[pl.ds(j, 16)])
                x_t[pl.ds(j, 16)] = 1.0 / (1.0 + jnp.exp(-z))
            return 0
        lax.fori_loop(0, NV // 5, upd_body, 0)
        plsc.subcore_barrier()

    for _ in range(NUM_PASSES):
        one_pass()

    lanes = lax.iota(jnp.int32, 16)

    def mean_body(i, acc):
        for u in range(5):
            j = i * 5 + u
            ids = lanes + j * 16
            msk = (ids % 10) == 0
            xv = x_t[pl.ds(j * 16, 16)]
            acc = acc + jnp.where(msk, xv, zeros16)
        return acc
    acc = lax.fori_loop(0, NV // 5, mean_body, zeros16)
    meanb = jnp.broadcast_to(jnp.sum(acc) * (1.0 / 1000.0), (16,))
    fcv = fcbuf[pl.ds(0, 16)]
    w00 = jnp.broadcast_to(fcv[0], (16,))
    w10 = jnp.broadcast_to(fcv[1], (16,))
    b0 = jnp.broadcast_to(fcv[2], (16,))
    b1 = jnp.broadcast_to(fcv[3], (16,))
    res = jnp.where(lanes == 0, meanb * w00 + b0,
                    jnp.where(lanes == 1, meanb * w10 + b1, zeros16))
    headbuf[0, pl.ds(0, 16)] = res
    pltpu.async_copy(headbuf.at[pl.ds(0, 1)], out_hbm.at[t], sem_x).wait()


def _fused(xf, packed, ew, ewm, thr, fcflat):
    mesh = plsc.VectorSubcoreMesh(core_axis_name="c", subcore_axis_name="s",
                                  num_cores=1)
    f = functools.partial(
        pl.kernel,
        out_type=jax.ShapeDtypeStruct((B, 1, 128), jnp.float32),
        mesh=mesh,
        compiler_params=pltpu.CompilerParams(needs_layout_passes=False),
        scratch_types=[
            pltpu.VMEM_SHARED((E,), jnp.float32),
            pltpu.VMEM_SHARED((16, 2, 16), jnp.float32),
            pltpu.VMEM((N,), jnp.float32),
            pltpu.VMEM((N,), jnp.float32),
            pltpu.VMEM((N,), jnp.float32),
            pltpu.VMEM((ES,), jnp.float32),
            pltpu.VMEM((ES,), jnp.float32),
            pltpu.VMEM((K,), jnp.int32),
            pltpu.VMEM((K,), jnp.int32),
            pltpu.VMEM((K,), jnp.float32),
            pltpu.VMEM((K,), jnp.float32),
            pltpu.VMEM((2, 16), jnp.float32),
            pltpu.VMEM((16, 2, 16), jnp.float32),
            pltpu.VMEM((16,), jnp.float32),
            pltpu.VMEM((1, 128), jnp.float32),
            pltpu.SemaphoreType.DMA,
            pltpu.SemaphoreType.DMA,
            pltpu.SemaphoreType.DMA,
            pltpu.SemaphoreType.DMA,
            pltpu.SemaphoreType.DMA,
        ],
    )(_fused_body)
    return f(xf, packed, ew, ewm, thr, fcflat)


def kernel(x, edge_index, edge_weight, edge_weight_multiplier,
           neuron_threshold, fc_w, fc_b):
    xf = x[:, 0]
    off = (jnp.arange(B, dtype=jnp.int32) * N)[:, None]
    srcl = edge_index[0].reshape(B, E) - off
    dstl = edge_index[1].reshape(B, E) - off
    packed = ((srcl << 14) | dstl).reshape(-1)
    fcflat = jnp.concatenate([fc_w[:, 0], fc_b,
                              jnp.zeros((12,), jnp.float32)])
    out = _fused(xf, packed, edge_weight, edge_weight_multiplier,
                 neuron_threshold, fcflat)
    return out[:, 0, :2]


# speedup vs baseline: 1.0460x; 1.0460x over previous
import functools

import jax
import jax.numpy as jnp
from jax import lax
from jax.experimental import pallas as pl
from jax.experimental.pallas import tpu as pltpu
from jax.experimental.pallas import tpu_sc as plsc

B = 16
N = 10000
E = 320000
NN = B * N
BE = B * E
NUM_PASSES = 4

K = 6400
CPW = K // 128
NWIN = E // K
NV = N // 16
ES = E // 16


def _fused_body(x_hbm, pk_hbm, ew_hbm, ewm_hbm, thr_hbm, fc_hbm, out_hbm,
                wc_sp, mm_sp,
                x_t, aggr_t, thr_t, stage, stage2,
                pk0, pk1, w0, w1,
                mmstage, mmall, fcbuf, headbuf,
                semS0, semS1, semW0, semW1, sem_x):
    t = lax.axis_index("s")
    tN = t * N

    pltpu.async_copy(x_hbm.at[pl.ds(tN, N)], x_t, sem_x).wait()
    pltpu.async_copy(thr_hbm.at[pl.ds(0, N)], thr_t, sem_x).wait()
    pltpu.async_copy(ew_hbm.at[pl.ds(t * ES, ES)], stage, sem_x).wait()
    pltpu.async_copy(ewm_hbm.at[pl.ds(t * ES, ES)], stage2, sem_x).wait()

    @plsc.parallel_loop(0, ES // 16, step=1, unroll=5)
    def wc_body(i):
        stage[pl.ds(i * 16, 16)] = (stage[pl.ds(i * 16, 16)]
                                    * stage2[pl.ds(i * 16, 16)])

    pltpu.async_copy(stage.at[pl.ds(0, ES)],
                     wc_sp.at[pl.ds(t * ES, ES)], sem_x).wait()
    pltpu.async_copy(fc_hbm.at[pl.ds(0, 16)], fcbuf, sem_x).wait()
    plsc.subcore_barrier()

    zeros16 = jnp.zeros((16,), jnp.float32)

    def edge_window(k, pkb, wb, semS, semW, prefetch):
        pltpu.make_async_copy(pk_hbm.at[pl.ds(0, K)], pkb, semS).wait()
        pltpu.make_async_copy(ew_hbm.at[pl.ds(0, K)], wb, semW).wait()

        @plsc.parallel_loop(0, K // 16, step=1, unroll=20)
        def vec_body(i):
            pk = pkb[pl.ds(i * 16, 16)]
            sl = lax.shift_right_logical(pk, 14)
            dl = lax.bitwise_and(pk, 0x3FFF)
            xs = plsc.load_gather(x_t, [sl])
            plsc.addupdate_scatter(aggr_t, [dl],
                                   xs * wb[pl.ds(i * 16, 16)])

        def fire():
            pltpu.async_copy(pk_hbm.at[pl.ds(t * E + (k + 2) * K, K)],
                             pkb, semS)
            pltpu.async_copy(wc_sp.at[pl.ds((k + 2) * K, K)], wb, semW)
        if prefetch is True:
            fire()
        elif prefetch is not False:
            pl.when(prefetch)(fire)

    def one_pass():
        def zero_body(i, _):
            for u in range(5):
                aggr_t[pl.ds((i * 5 + u) * 16, 16)] = zeros16
            return 0
        lax.fori_loop(0, NV // 5, zero_body, 0)

        pltpu.async_copy(pk_hbm.at[pl.ds(t * E, K)], pk0, semS0)
        pltpu.async_copy(wc_sp.at[pl.ds(0, K)], w0, semW0)
        pltpu.async_copy(pk_hbm.at[pl.ds(t * E + K, K)], pk1, semS1)
        pltpu.async_copy(wc_sp.at[pl.ds(K, K)], w1, semW1)

        edge_window(0, pk0, w0, semS0, semW0, True)
        edge_window(1, pk1, w1, semS1, semW1, True)

        def pair_body(m, _):
            pf = m < (NWIN // 2 - 1)
            edge_window(2 * m, pk0, w0, semS0, semW0, pf)
            edge_window(2 * m + 1, pk1, w1, semS1, semW1, pf)
            return 0
        lax.fori_loop(1, NWIN // 2, pair_body, 0)

        def mm_body(i, carry):
            mn, mx = carry
            for u in range(5):
                a = aggr_t[pl.ds((i * 5 + u) * 16, 16)]
                mn = jnp.minimum(mn, a)
                mx = jnp.maximum(mx, a)
            return mn, mx
        mnv, mxv = lax.fori_loop(
            0, NV // 5, mm_body,
            (jnp.full((16,), jnp.inf, jnp.float32),
             jnp.full((16,), -jnp.inf, jnp.float32)))
        mmstage[0, pl.ds(0, 16)] = mnv
        mmstage[1, pl.ds(0, 16)] = mxv
        pltpu.async_copy(mmstage.at[pl.ds(0, 2)], mm_sp.at[t], sem_x).wait()
        plsc.subcore_barrier()
        pltpu.async_copy(mm_sp.at[pl.ds(0, 16)], mmall, sem_x).wait()
        gmn = jnp.full((16,), jnp.inf, jnp.float32)
        gmx = jnp.full((16,), -jnp.inf, jnp.float32)
        for r in range(16):
            gmn = jnp.minimum(gmn, mmall[r, 0, pl.ds(0, 16)])
            gmx = jnp.maximum(gmx, mmall[r, 1, pl.ds(0, 16)])
        mnb = jnp.broadcast_to(jnp.min(gmn), (16,))
        mxb = jnp.broadcast_to(jnp.max(gmx), (16,))
        invb = 1.0 / (mxb - mnb)

        def upd_body(i, _):
            for u in range(5):
                j = (i * 5 + u) * 16
                a = aggr_t[pl.ds(j, 16)]
                z = (a - mnb) * invb - jnp.abs(thr_t[pl.ds(j, 16)])
                x_t[pl.ds(j, 16)] = 1.0 / (1.0 + jnp.exp(-z))
            return 0
        lax.fori_loop(0, NV // 5, upd_body, 0)
        plsc.subcore_barrier()

    for _ in range(NUM_PASSES):
        one_pass()

    lanes = lax.iota(jnp.int32, 16)

    def mean_body(i, acc):
        for u in range(5):
            j = i * 5 + u
            ids = lanes + j * 16
            msk = (ids % 10) == 0
            xv = x_t[pl.ds(j * 16, 16)]
            acc = acc + jnp.where(msk, xv, zeros16)
        return acc
    acc = lax.fori_loop(0, NV // 5, mean_body, zeros16)
    meanb = jnp.broadcast_to(jnp.sum(acc) * (1.0 / 1000.0), (16,))
    fcv = fcbuf[pl.ds(0, 16)]
    w00 = jnp.broadcast_to(fcv[0], (16,))
    w10 = jnp.broadcast_to(fcv[1], (16,))
    b0 = jnp.broadcast_to(fcv[2], (16,))
    b1 = jnp.broadcast_to(fcv[3], (16,))
    res = jnp.where(lanes == 0, meanb * w00 + b0,
                    jnp.where(lanes == 1, meanb * w10 + b1, zeros16))
    headbuf[0, pl.ds(0, 16)] = res
    pltpu.async_copy(headbuf.at[pl.ds(0, 1)], out_hbm.at[t], sem_x).wait()


def _fused(xf, packed, ew, ewm, thr, fcflat):
    mesh = plsc.VectorSubcoreMesh(core_axis_name="c", subcore_axis_name="s",
                                  num_cores=1)
    f = functools.partial(
        pl.kernel,
        out_type=jax.ShapeDtypeStruct((B, 1, 128), jnp.float32),
        mesh=mesh,
        compiler_params=pltpu.CompilerParams(needs_layout_passes=False),
        scratch_types=[
            pltpu.VMEM_SHARED((E,), jnp.float32),
            pltpu.VMEM_SHARED((16, 2, 16), jnp.float32),
            pltpu.VMEM((N,), jnp.float32),
            pltpu.VMEM((N,), jnp.float32),
            pltpu.VMEM((N,), jnp.float32),
            pltpu.VMEM((ES,), jnp.float32),
            pltpu.VMEM((ES,), jnp.float32),
            pltpu.VMEM((K,), jnp.int32),
            pltpu.VMEM((K,), jnp.int32),
            pltpu.VMEM((K,), jnp.float32),
            pltpu.VMEM((K,), jnp.float32),
            pltpu.VMEM((2, 16), jnp.float32),
            pltpu.VMEM((16, 2, 16), jnp.float32),
            pltpu.VMEM((16,), jnp.float32),
            pltpu.VMEM((1, 128), jnp.float32),
            pltpu.SemaphoreType.DMA,
            pltpu.SemaphoreType.DMA,
            pltpu.SemaphoreType.DMA,
            pltpu.SemaphoreType.DMA,
            pltpu.SemaphoreType.DMA,
        ],
    )(_fused_body)
    return f(xf, packed, ew, ewm, thr, fcflat)


def kernel(x, edge_index, edge_weight, edge_weight_multiplier,
           neuron_threshold, fc_w, fc_b):
    xf = x[:, 0]
    off = (jnp.arange(B, dtype=jnp.int32) * N)[:, None]
    srcl = edge_index[0].reshape(B, E) - off
    dstl = edge_index[1].reshape(B, E) - off
    packed = ((srcl << 14) | dstl).reshape(-1)
    fcflat = jnp.concatenate([fc_w[:, 0], fc_b,
                              jnp.zeros((12,), jnp.float32)])
    out = _fused(xf, packed, edge_weight, edge_weight_multiplier,
                 neuron_threshold, fcflat)
    return out[:, 0, :2]
